# baseline (speedup 1.0000x reference)
import functools

import jax
import jax.numpy as jnp
from jax import lax
from jax.experimental import pallas as pl
from jax.experimental.pallas import tpu as pltpu
from jax.experimental.pallas import tpu_sc as plsc

N = 10000
D = 128
E = 320000
EPS = 1e-5

NC, NS, L = 2, 16, 16
NW = NC * NS
EPT = E // NW
K = 125
NCH = EPT // K
STG = 40
NST = NCH // STG
NPAD = 10240
RPT = NPAD // NS

_mesh = plsc.VectorSubcoreMesh(
    core_axis_name="c", subcore_axis_name="s", num_cores=NC, num_subcores=NS
)


def _seg_body(x_hbm, src_hbm, dst_hbm, zeros_hbm, parts,
              acc_sh, src_v, dst_v, rows_a, rows_b, ga, gb, sa, sb):
    c = lax.axis_index("c")
    s = lax.axis_index("s")
    wid = c * NS + s
    r0 = s * RPT

    pltpu.sync_copy(zeros_hbm, acc_sh.at[pl.ds(r0, RPT)])
    plsc.subcore_barrier()

    for p in range(NST):
        base = wid * NCH + p * STG
        pltpu.sync_copy(src_hbm.at[pl.ds(base, STG)], src_v)
        pltpu.sync_copy(dst_hbm.at[pl.ds(base, STG)], dst_v)

        pltpu.async_copy(x_hbm.at[src_v.at[0]], rows_a, ga)

        def chunk(j, carry):
            nxt = j + 1

            @pl.when(j % 2 == 0)
            def _even():
                @pl.when(nxt < STG)
                def _():
                    @pl.when(j >= 2)
                    def _():
                        pltpu.make_async_copy(
                            rows_b, acc_sh.at[dst_v.at[j]], sb).wait()
                    pltpu.async_copy(x_hbm.at[src_v.at[nxt]], rows_b, gb)
                pltpu.make_async_copy(x_hbm.at[src_v.at[j]], rows_a, ga).wait()
                pltpu.async_copy(rows_a, acc_sh.at[dst_v.at[j]], sa, add=True)

            @pl.when(j % 2 == 1)
            def _odd():
                @pl.when(nxt < STG)
                def _():
                    pltpu.make_async_copy(
                        rows_a, acc_sh.at[dst_v.at[j]], sa).wait()
                    pltpu.async_copy(x_hbm.at[src_v.at[nxt]], rows_a, ga)
                pltpu.make_async_copy(x_hbm.at[src_v.at[j]], rows_b, gb).wait()
                pltpu.async_copy(rows_b, acc_sh.at[dst_v.at[j]], sb, add=True)

            return carry

        lax.fori_loop(0, STG, chunk, 0)
        pltpu.make_async_copy(rows_a, acc_sh.at[dst_v.at[0]], sa).wait()
        pltpu.make_async_copy(rows_b, acc_sh.at[dst_v.at[0]], sb).wait()

    plsc.subcore_barrier()
    pltpu.sync_copy(acc_sh.at[pl.ds(r0, RPT)], parts.at[c, pl.ds(r0, RPT)])


_sc_seg = functools.partial(
    pl.kernel,
    out_type=jax.ShapeDtypeStruct((NC, NPAD, D), jnp.float32),
    mesh=_mesh,
    scratch_types=[
        pltpu.VMEM_SHARED((NPAD, D), jnp.float32),
        pltpu.VMEM((STG, K), jnp.int32),
        pltpu.VMEM((STG, K), jnp.int32),
        pltpu.VMEM((K, D), jnp.float32),
        pltpu.VMEM((K, D), jnp.float32),
        pltpu.SemaphoreType.DMA,
        pltpu.SemaphoreType.DMA,
        pltpu.SemaphoreType.DMA,
        pltpu.SemaphoreType.DMA,
    ],
)(_seg_body)


DW = D


def _deg_body(dst_hbm, zeros_hbm, ones_hbm, degp,
              deg_sh, dst_v, ones_v, sem):
    c = lax.axis_index("c")
    s = lax.axis_index("s")
    wid = c * NS + s
    r0 = s * RPT

    pltpu.sync_copy(zeros_hbm, deg_sh.at[pl.ds(r0, RPT)])
    pltpu.sync_copy(ones_hbm, ones_v)
    plsc.subcore_barrier()

    for p in range(NST):
        base = wid * NCH + p * STG
        pltpu.sync_copy(dst_hbm.at[pl.ds(base, STG)], dst_v)

        def group(g, carry):
            for t in range(8):
                pltpu.async_copy(
                    ones_v, deg_sh.at[dst_v.at[g * 8 + t]], sem, add=True)
            for t in range(8):
                pltpu.make_async_copy(ones_v, deg_sh.at[dst_v.at[0]], sem).wait()
            return carry

        lax.fori_loop(0, STG // 8, group, 0)

    plsc.subcore_barrier()
    pltpu.sync_copy(deg_sh.at[pl.ds(r0, RPT)], degp.at[c, pl.ds(r0, RPT)])


_sc_deg = functools.partial(
    pl.kernel,
    out_type=jax.ShapeDtypeStruct((NC, NPAD, DW), jnp.float32),
    mesh=_mesh,
    scratch_types=[
        pltpu.VMEM_SHARED((NPAD, DW), jnp.float32),
        pltpu.VMEM((STG, K), jnp.int32),
        pltpu.VMEM((K, DW), jnp.float32),
        pltpu.SemaphoreType.DMA,
    ],
)(_deg_body)


_BN = 1000


def _tc_z_body(h_ref, wrT_ref, b_ref, z_ref):
    z_ref[...] = jnp.dot(h_ref[...], wrT_ref[...],
                         preferred_element_type=jnp.float32) + b_ref[...]


def _tc_z(h, wrT, b):
    return pl.pallas_call(
        _tc_z_body,
        grid=(N // _BN,),
        in_specs=[
            pl.BlockSpec((_BN, D), lambda i: (i, 0)),
            pl.BlockSpec((D, D), lambda i: (0, 0)),
            pl.BlockSpec((1, D), lambda i: (0, 0)),
        ],
        out_specs=pl.BlockSpec((_BN, D), lambda i: (i, 0)),
        out_shape=jax.ShapeDtypeStruct((N, D), jnp.float32),
    )(h, wrT, b)


def _tc_h_body(relu, parts_ref, degp_ref, z_ref, wlT_ref, out_ref):
    ssum = parts_ref[0] + parts_ref[1]
    deg = degp_ref[0, :, 0:1] + degp_ref[1, :, 0:1]
    agg = ssum / jnp.maximum(deg, 1.0)
    out = jnp.dot(agg, wlT_ref[...],
                  preferred_element_type=jnp.float32) + z_ref[...]
    if relu:
        out = jnp.maximum(out, 0.0)
    out_ref[...] = out


def _tc_h(parts, degp, z, wlT, relu):
    return pl.pallas_call(
        functools.partial(_tc_h_body, relu),
        grid=(N // _BN,),
        in_specs=[
            pl.BlockSpec((NC, _BN, D), lambda i: (0, i, 0)),
            pl.BlockSpec((NC, _BN, DW), lambda i: (0, i, 0)),
            pl.BlockSpec((_BN, D), lambda i: (i, 0)),
            pl.BlockSpec((D, D), lambda i: (0, 0)),
        ],
        out_specs=pl.BlockSpec((_BN, D), lambda i: (i, 0)),
        out_shape=jax.ShapeDtypeStruct((N, D), jnp.float32),
    )(parts, degp, z, wlT)


def kernel(x, edge_index, Wl1, bl1, Wr1, Wl2, bl2, Wr2, Wl3, bl3, Wr3,
           g1, be1, g2, be2):
    src = edge_index[0].astype(jnp.int32).reshape(NW * NCH, K)
    dst = edge_index[1].astype(jnp.int32).reshape(NW * NCH, K)
    zeros = jnp.zeros((RPT, D), jnp.float32)
    zerosw = jnp.zeros((RPT, DW), jnp.float32)
    ones = jnp.ones((K, DW), jnp.float32)

    gs1 = g1 / jnp.sqrt(1.0 + EPS)
    gs2 = g2 / jnp.sqrt(1.0 + EPS)
    wlT1 = Wl1.T * gs1[None, :]
    wrT1 = Wr1.T * gs1[None, :]
    b1 = (bl1 * gs1 + be1)[None, :]
    wlT2 = Wl2.T * gs2[None, :]
    wrT2 = Wr2.T * gs2[None, :]
    b2 = (bl2 * gs2 + be2)[None, :]
    wlT3 = Wl3.T
    wrT3 = Wr3.T
    b3 = bl3[None, :]

    z1 = _tc_z(x, wrT1, b1)
    degp = _sc_deg(dst, zerosw, ones)
    parts = _sc_seg(x, src, dst, zeros)
    h = _tc_h(parts, degp, z1, wlT1, True)
    z2 = _tc_z(h, wrT2, b2)
    parts = _sc_seg(h, src, dst, zeros)
    h = _tc_h(parts, degp, z2, wlT2, True)
    z3 = _tc_z(h, wrT3, b3)
    parts = _sc_seg(h, src, dst, zeros)
    h = _tc_h(parts, degp, z3, wlT3, False)
    return h


# speedup vs baseline: 1.0102x; 1.0102x over previous
import functools

import jax
import jax.numpy as jnp
from jax import lax
from jax.experimental import pallas as pl
from jax.experimental.pallas import tpu as pltpu
from jax.experimental.pallas import tpu_sc as plsc

N = 10000
D = 128
E = 320000
EPS = 1e-5

NC, NS, L = 2, 16, 16
NW = NC * NS
EPT = E // NW
K = 125
NCH = EPT // K
STG = 40
NST = NCH // STG
NPAD = 10240
RPT = NPAD // NS

_mesh = plsc.VectorSubcoreMesh(
    core_axis_name="c", subcore_axis_name="s", num_cores=NC, num_subcores=NS
)


def _seg_body(x_hbm, src_hbm, dst_hbm, zeros_hbm, parts,
              acc_sh, src_v, dst_v, rows_a, rows_b, ga, gb, sa, sb):
    c = lax.axis_index("c")
    s = lax.axis_index("s")
    wid = c * NS + s
    r0 = s * RPT

    pltpu.sync_copy(zeros_hbm, acc_sh.at[pl.ds(r0, RPT)])
    plsc.subcore_barrier()

    for p in range(NST):
        base = wid * NCH + p * STG
        pltpu.sync_copy(src_hbm.at[pl.ds(base, STG)], src_v)
        pltpu.sync_copy(dst_hbm.at[pl.ds(base, STG)], dst_v)

        pltpu.async_copy(x_hbm.at[src_v.at[0]], rows_a, ga)

        def chunk(j, carry):
            nxt = j + 1

            @pl.when(j % 2 == 0)
            def _even():
                @pl.when(nxt < STG)
                def _():
                    @pl.when(j >= 2)
                    def _():
                        pltpu.make_async_copy(
                            rows_b, acc_sh.at[dst_v.at[j]], sb).wait()
                    pltpu.async_copy(x_hbm.at[src_v.at[nxt]], rows_b, gb)
                pltpu.make_async_copy(x_hbm.at[src_v.at[j]], rows_a, ga).wait()
                pltpu.async_copy(rows_a, acc_sh.at[dst_v.at[j]], sa, add=True)

            @pl.when(j % 2 == 1)
            def _odd():
                @pl.when(nxt < STG)
                def _():
                    pltpu.make_async_copy(
                        rows_a, acc_sh.at[dst_v.at[j]], sa).wait()
                    pltpu.async_copy(x_hbm.at[src_v.at[nxt]], rows_a, ga)
                pltpu.make_async_copy(x_hbm.at[src_v.at[j]], rows_b, gb).wait()
                pltpu.async_copy(rows_b, acc_sh.at[dst_v.at[j]], sb, add=True)

            return carry

        lax.fori_loop(0, STG, chunk, 0)
        pltpu.make_async_copy(rows_a, acc_sh.at[dst_v.at[0]], sa).wait()
        pltpu.make_async_copy(rows_b, acc_sh.at[dst_v.at[0]], sb).wait()

    plsc.subcore_barrier()
    pltpu.sync_copy(acc_sh.at[pl.ds(r0, RPT)], parts.at[c, pl.ds(r0, RPT)])


_sc_seg = functools.partial(
    pl.kernel,
    out_type=jax.ShapeDtypeStruct((NC, NPAD, D), jnp.float32),
    mesh=_mesh,
    scratch_types=[
        pltpu.VMEM_SHARED((NPAD, D), jnp.float32),
        pltpu.VMEM((STG, K), jnp.int32),
        pltpu.VMEM((STG, K), jnp.int32),
        pltpu.VMEM((K, D), jnp.float32),
        pltpu.VMEM((K, D), jnp.float32),
        pltpu.SemaphoreType.DMA,
        pltpu.SemaphoreType.DMA,
        pltpu.SemaphoreType.DMA,
        pltpu.SemaphoreType.DMA,
    ],
)(_seg_body)


DW = D


def _deg_body(dst_hbm, zeros_hbm, ones_hbm, degp,
              deg_sh, dst_v, ones_v, sem):
    c = lax.axis_index("c")
    s = lax.axis_index("s")
    wid = c * NS + s
    r0 = s * RPT

    pltpu.sync_copy(zeros_hbm, deg_sh.at[pl.ds(r0, RPT)])
    pltpu.sync_copy(ones_hbm, ones_v)
    plsc.subcore_barrier()

    for p in range(NST):
        base = wid * NCH + p * STG
        pltpu.sync_copy(dst_hbm.at[pl.ds(base, STG)], dst_v)

        def group(g, carry):
            for t in range(8):
                pltpu.async_copy(
                    ones_v, deg_sh.at[dst_v.at[g * 8 + t]], sem, add=True)
            for t in range(8):
                pltpu.make_async_copy(ones_v, deg_sh.at[dst_v.at[0]], sem).wait()
            return carry

        lax.fori_loop(0, STG // 8, group, 0)

    plsc.subcore_barrier()
    pltpu.sync_copy(deg_sh.at[pl.ds(r0, RPT)], degp.at[c, pl.ds(r0, RPT)])


_sc_deg = functools.partial(
    pl.kernel,
    out_type=jax.ShapeDtypeStruct((NC, NPAD, DW), jnp.float32),
    mesh=_mesh,
    scratch_types=[
        pltpu.VMEM_SHARED((NPAD, DW), jnp.float32),
        pltpu.VMEM((STG, K), jnp.int32),
        pltpu.VMEM((K, DW), jnp.float32),
        pltpu.SemaphoreType.DMA,
    ],
)(_deg_body)


def _tc_body(relu, parts_ref, degp_ref, h_ref, wlT_ref, wrT_ref, b_ref, out_ref):
    ssum = parts_ref[0] + parts_ref[1]
    deg = degp_ref[0, :, 0:1] + degp_ref[1, :, 0:1]
    agg = ssum / jnp.maximum(deg, 1.0)
    out = jnp.dot(agg, wlT_ref[...], preferred_element_type=jnp.float32)
    out = out + jnp.dot(h_ref[...], wrT_ref[...], preferred_element_type=jnp.float32)
    out = out + b_ref[...]
    if relu:
        out = jnp.maximum(out, 0.0)
    out_ref[...] = out


_BN = 1000


def _tc_layer(parts, degp, h, wlT, wrT, b, relu):
    grid = (N // _BN,)
    return pl.pallas_call(
        functools.partial(_tc_body, relu),
        grid=grid,
        in_specs=[
            pl.BlockSpec((NC, _BN, D), lambda i: (0, i, 0)),
            pl.BlockSpec((NC, _BN, DW), lambda i: (0, i, 0)),
            pl.BlockSpec((_BN, D), lambda i: (i, 0)),
            pl.BlockSpec((D, D), lambda i: (0, 0)),
            pl.BlockSpec((D, D), lambda i: (0, 0)),
            pl.BlockSpec((1, D), lambda i: (0, 0)),
        ],
        out_specs=pl.BlockSpec((_BN, D), lambda i: (i, 0)),
        out_shape=jax.ShapeDtypeStruct((N, D), jnp.float32),
    )(parts, degp, h, wlT, wrT, b)


def kernel(x, edge_index, Wl1, bl1, Wr1, Wl2, bl2, Wr2, Wl3, bl3, Wr3,
           g1, be1, g2, be2):
    src = edge_index[0].astype(jnp.int32).reshape(NW * NCH, K)
    dst = edge_index[1].astype(jnp.int32).reshape(NW * NCH, K)
    zeros = jnp.zeros((RPT, D), jnp.float32)
    zerosw = jnp.zeros((RPT, DW), jnp.float32)
    ones = jnp.ones((K, DW), jnp.float32)

    gs1 = g1 / jnp.sqrt(1.0 + EPS)
    gs2 = g2 / jnp.sqrt(1.0 + EPS)
    wlT1 = Wl1.T * gs1[None, :]
    wrT1 = Wr1.T * gs1[None, :]
    b1 = (bl1 * gs1 + be1)[None, :]
    wlT2 = Wl2.T * gs2[None, :]
    wrT2 = Wr2.T * gs2[None, :]
    b2 = (bl2 * gs2 + be2)[None, :]
    wlT3 = Wl3.T
    wrT3 = Wr3.T
    b3 = bl3[None, :]

    degp = _sc_deg(dst, zerosw, ones)
    parts = _sc_seg(x, src, dst, zeros)
    h = _tc_layer(parts, degp, x, wlT1, wrT1, b1, True)
    parts = _sc_seg(h, src, dst, zeros)
    h = _tc_layer(parts, degp, h, wlT2, wrT2, b2, True)
    parts = _sc_seg(h, src, dst, zeros)
    h = _tc_layer(parts, degp, h, wlT3, wrT3, b3, False)
    return h


# speedup vs baseline: 1.0222x; 1.0119x over previous
import functools

import jax
import jax.numpy as jnp
from jax import lax
from jax.experimental import pallas as pl
from jax.experimental.pallas import tpu as pltpu
from jax.experimental.pallas import tpu_sc as plsc

N = 10000
D = 128
E = 320000
EPS = 1e-5

NC, NS, L = 2, 16, 16
NW = NC * NS
EPT = E // NW
K = 125
NCH = EPT // K
STG = 40
NST = NCH // STG
NPAD = 10240
RPT = NPAD // NS

_mesh = plsc.VectorSubcoreMesh(
    core_axis_name="c", subcore_axis_name="s", num_cores=NC, num_subcores=NS
)


def _seg_core(x_hbm, src_hbm, dst_hbm, out_hbm,
              acc_sh, src_v, dst_v, rows_a, rows_b, ga, gb, sa, sb,
              c, s, wid, r0):
    for p in range(NST):
        base = wid * NCH + p * STG
        pltpu.sync_copy(src_hbm.at[pl.ds(base, STG)], src_v)
        pltpu.sync_copy(dst_hbm.at[pl.ds(base, STG)], dst_v)

        pltpu.async_copy(x_hbm.at[src_v.at[0]], rows_a, ga)

        def chunk(j, carry):
            nxt = j + 1

            @pl.when(j % 2 == 0)
            def _even():
                @pl.when(nxt < STG)
                def _():
                    @pl.when(j >= 2)
                    def _():
                        pltpu.make_async_copy(
                            rows_b, acc_sh.at[dst_v.at[j]], sb).wait()
                    pltpu.async_copy(x_hbm.at[src_v.at[nxt]], rows_b, gb)
                pltpu.make_async_copy(x_hbm.at[src_v.at[j]], rows_a, ga).wait()
                pltpu.async_copy(rows_a, acc_sh.at[dst_v.at[j]], sa, add=True)

            @pl.when(j % 2 == 1)
            def _odd():
                @pl.when(nxt < STG)
                def _():
                    pltpu.make_async_copy(
                        rows_a, acc_sh.at[dst_v.at[j]], sa).wait()
                    pltpu.async_copy(x_hbm.at[src_v.at[nxt]], rows_a, ga)
                pltpu.make_async_copy(x_hbm.at[src_v.at[j]], rows_b, gb).wait()
                pltpu.async_copy(rows_b, acc_sh.at[dst_v.at[j]], sb, add=True)

            return carry

        lax.fori_loop(0, STG, chunk, 0)
        pltpu.make_async_copy(rows_a, acc_sh.at[dst_v.at[0]], sa).wait()
        pltpu.make_async_copy(rows_b, acc_sh.at[dst_v.at[0]], sb).wait()

    plsc.subcore_barrier()
    pltpu.sync_copy(acc_sh.at[pl.ds(r0, RPT)], out_hbm.at[c, pl.ds(r0, RPT)])


def _seg_body(x_hbm, src_hbm, dst_hbm, zeros_hbm, parts,
              acc_sh, src_v, dst_v, rows_a, rows_b, ga, gb, sa, sb):
    c = lax.axis_index("c")
    s = lax.axis_index("s")
    wid = c * NS + s
    r0 = s * RPT

    pltpu.sync_copy(zeros_hbm, acc_sh.at[pl.ds(r0, RPT)])
    plsc.subcore_barrier()
    _seg_core(x_hbm, src_hbm, dst_hbm, parts,
              acc_sh, src_v, dst_v, rows_a, rows_b, ga, gb, sa, sb,
              c, s, wid, r0)


_SEG_SCRATCH = [
    pltpu.VMEM_SHARED((NPAD, D), jnp.float32),
    pltpu.VMEM((STG, K), jnp.int32),
    pltpu.VMEM((STG, K), jnp.int32),
    pltpu.VMEM((K, D), jnp.float32),
    pltpu.VMEM((K, D), jnp.float32),
    pltpu.SemaphoreType.DMA,
    pltpu.SemaphoreType.DMA,
    pltpu.SemaphoreType.DMA,
    pltpu.SemaphoreType.DMA,
]

_sc_seg = functools.partial(
    pl.kernel,
    out_type=jax.ShapeDtypeStruct((NC, NPAD, D), jnp.float32),
    mesh=_mesh,
    scratch_types=list(_SEG_SCRATCH),
)(_seg_body)


DW = D


def _segdeg_body(x_hbm, src_hbm, dst_hbm, zeros_hbm, ones_hbm, degp, parts,
                 acc_sh, src_v, dst_v, rows_a, rows_b, ga, gb, sa, sb):
    c = lax.axis_index("c")
    s = lax.axis_index("s")
    wid = c * NS + s
    r0 = s * RPT

    pltpu.sync_copy(zeros_hbm, acc_sh.at[pl.ds(r0, RPT)])
    pltpu.sync_copy(ones_hbm, rows_a)
    plsc.subcore_barrier()

    for p in range(NST):
        base = wid * NCH + p * STG
        pltpu.sync_copy(dst_hbm.at[pl.ds(base, STG)], dst_v)

        def group(g, carry):
            for t in range(8):
                pltpu.async_copy(
                    rows_a, acc_sh.at[dst_v.at[g * 8 + t]], sa, add=True)
            for t in range(8):
                pltpu.make_async_copy(rows_a, acc_sh.at[dst_v.at[0]], sa).wait()
            return carry

        lax.fori_loop(0, STG // 8, group, 0)

    plsc.subcore_barrier()
    pltpu.sync_copy(acc_sh.at[pl.ds(r0, RPT)], degp.at[c, pl.ds(r0, RPT)])
    plsc.subcore_barrier()
    pltpu.sync_copy(zeros_hbm, acc_sh.at[pl.ds(r0, RPT)])
    plsc.subcore_barrier()
    _seg_core(x_hbm, src_hbm, dst_hbm, parts,
              acc_sh, src_v, dst_v, rows_a, rows_b, ga, gb, sa, sb,
              c, s, wid, r0)


_sc_segdeg = functools.partial(
    pl.kernel,
    out_type=[
        jax.ShapeDtypeStruct((NC, NPAD, DW), jnp.float32),
        jax.ShapeDtypeStruct((NC, NPAD, D), jnp.float32),
    ],
    mesh=_mesh,
    scratch_types=list(_SEG_SCRATCH),
)(_segdeg_body)


def _tc_body(relu, parts_ref, degp_ref, h_ref, wlT_ref, wrT_ref, b_ref, out_ref):
    ssum = parts_ref[0] + parts_ref[1]
    deg = degp_ref[0, :, 0:1] + degp_ref[1, :, 0:1]
    agg = ssum / jnp.maximum(deg, 1.0)
    out = jnp.dot(agg, wlT_ref[...], preferred_element_type=jnp.float32)
    out = out + jnp.dot(h_ref[...], wrT_ref[...], preferred_element_type=jnp.float32)
    out = out + b_ref[...]
    if relu:
        out = jnp.maximum(out, 0.0)
    out_ref[...] = out


_BN = 1000


def _tc_layer(parts, degp, h, wlT, wrT, b, relu):
    grid = (N // _BN,)
    return pl.pallas_call(
        functools.partial(_tc_body, relu),
        grid=grid,
        in_specs=[
            pl.BlockSpec((NC, _BN, D), lambda i: (0, i, 0)),
            pl.BlockSpec((NC, _BN, DW), lambda i: (0, i, 0)),
            pl.BlockSpec((_BN, D), lambda i: (i, 0)),
            pl.BlockSpec((D, D), lambda i: (0, 0)),
            pl.BlockSpec((D, D), lambda i: (0, 0)),
            pl.BlockSpec((1, D), lambda i: (0, 0)),
        ],
        out_specs=pl.BlockSpec((_BN, D), lambda i: (i, 0)),
        out_shape=jax.ShapeDtypeStruct((N, D), jnp.float32),
    )(parts, degp, h, wlT, wrT, b)


def kernel(x, edge_index, Wl1, bl1, Wr1, Wl2, bl2, Wr2, Wl3, bl3, Wr3,
           g1, be1, g2, be2):
    src = edge_index[0].astype(jnp.int32).reshape(NW * NCH, K)
    dst = edge_index[1].astype(jnp.int32).reshape(NW * NCH, K)
    zeros = jnp.zeros((RPT, D), jnp.float32)
    ones = jnp.ones((K, D), jnp.float32)

    gs1 = g1 / jnp.sqrt(1.0 + EPS)
    gs2 = g2 / jnp.sqrt(1.0 + EPS)
    wlT1 = Wl1.T * gs1[None, :]
    wrT1 = Wr1.T * gs1[None, :]
    b1 = (bl1 * gs1 + be1)[None, :]
    wlT2 = Wl2.T * gs2[None, :]
    wrT2 = Wr2.T * gs2[None, :]
    b2 = (bl2 * gs2 + be2)[None, :]
    wlT3 = Wl3.T
    wrT3 = Wr3.T
    b3 = bl3[None, :]

    degp, parts = _sc_segdeg(x, src, dst, zeros, ones)
    h = _tc_layer(parts, degp, x, wlT1, wrT1, b1, True)
    parts = _sc_seg(h, src, dst, zeros)
    h = _tc_layer(parts, degp, h, wlT2, wrT2, b2, True)
    parts = _sc_seg(h, src, dst, zeros)
    h = _tc_layer(parts, degp, h, wlT3, wrT3, b3, False)
    return h


# speedup vs baseline: 1.0385x; 1.0159x over previous
import functools

import jax
import jax.numpy as jnp
from jax import lax
from jax.experimental import pallas as pl
from jax.experimental.pallas import tpu as pltpu
from jax.experimental.pallas import tpu_sc as plsc

N = 10000
D = 128
E = 320000
EPS = 1e-5

NC, NS, L = 2, 16, 16
NW = NC * NS
EPT = E // NW
K = 125
NCH = EPT // K
STG = 40
NST = NCH // STG
NPAD = 10240
RPT = NPAD // NS

_mesh = plsc.VectorSubcoreMesh(
    core_axis_name="c", subcore_axis_name="s", num_cores=NC, num_subcores=NS
)


def _seg_core(x_hbm, src_hbm, dst_hbm, out_hbm,
              acc_sh, src_v, dst_v, rows_a, rows_b, ga, gb, sa, sb,
              c, s, wid, r0):
    for p in range(NST):
        base = wid * NCH + p * STG
        pltpu.sync_copy(src_hbm.at[pl.ds(base, STG)], src_v)
        pltpu.sync_copy(dst_hbm.at[pl.ds(base, STG)], dst_v)

        pltpu.async_copy(x_hbm.at[src_v.at[0]], rows_a, ga)

        def chunk(j, carry):
            nxt = j + 1

            @pl.when(j % 2 == 0)
            def _even():
                @pl.when(nxt < STG)
                def _():
                    @pl.when(j >= 2)
                    def _():
                        pltpu.make_async_copy(
                            rows_b, acc_sh.at[dst_v.at[j]], sb).wait()
                    pltpu.async_copy(x_hbm.at[src_v.at[nxt]], rows_b, gb)
                pltpu.make_async_copy(x_hbm.at[src_v.at[j]], rows_a, ga).wait()
                pltpu.async_copy(rows_a, acc_sh.at[dst_v.at[j]], sa, add=True)

            @pl.when(j % 2 == 1)
            def _odd():
                @pl.when(nxt < STG)
                def _():
                    pltpu.make_async_copy(
                        rows_a, acc_sh.at[dst_v.at[j]], sa).wait()
                    pltpu.async_copy(x_hbm.at[src_v.at[nxt]], rows_a, ga)
                pltpu.make_async_copy(x_hbm.at[src_v.at[j]], rows_b, gb).wait()
                pltpu.async_copy(rows_b, acc_sh.at[dst_v.at[j]], sb, add=True)

            return carry

        lax.fori_loop(0, STG, chunk, 0)
        pltpu.make_async_copy(rows_a, acc_sh.at[dst_v.at[0]], sa).wait()
        pltpu.make_async_copy(rows_b, acc_sh.at[dst_v.at[0]], sb).wait()

    plsc.subcore_barrier()
    pltpu.sync_copy(acc_sh.at[pl.ds(r0, RPT)], out_hbm.at[c, pl.ds(r0, RPT)])


def _seg_body(x_hbm, src_hbm, dst_hbm, zeros_hbm, parts,
              acc_sh, src_v, dst_v, rows_a, rows_b, ga, gb, sa, sb):
    c = lax.axis_index("c")
    s = lax.axis_index("s")
    wid = c * NS + s
    r0 = s * RPT

    pltpu.sync_copy(zeros_hbm, acc_sh.at[pl.ds(r0, RPT)])
    plsc.subcore_barrier()
    _seg_core(x_hbm, src_hbm, dst_hbm, parts,
              acc_sh, src_v, dst_v, rows_a, rows_b, ga, gb, sa, sb,
              c, s, wid, r0)


_SEG_SCRATCH = [
    pltpu.VMEM_SHARED((NPAD, D), jnp.float32),
    pltpu.VMEM((STG, K), jnp.int32),
    pltpu.VMEM((STG, K), jnp.int32),
    pltpu.VMEM((K, D), jnp.float32),
    pltpu.VMEM((K, D), jnp.float32),
    pltpu.SemaphoreType.DMA,
    pltpu.SemaphoreType.DMA,
    pltpu.SemaphoreType.DMA,
    pltpu.SemaphoreType.DMA,
]

_sc_seg = functools.partial(
    pl.kernel,
    out_type=jax.ShapeDtypeStruct((NC, NPAD, D), jnp.float32),
    mesh=_mesh,
    scratch_types=list(_SEG_SCRATCH),
)(_seg_body)


DW = D


def _segdeg_body(x_hbm, src_hbm, dst_hbm, zeros_hbm, ones_hbm, degp, parts,
                 acc_sh, src_v, dst_v, rows_a, rows_b, ga, gb, sa, sb):
    c = lax.axis_index("c")
    s = lax.axis_index("s")
    wid = c * NS + s
    r0 = s * RPT

    pltpu.sync_copy(zeros_hbm, acc_sh.at[pl.ds(r0, RPT)])
    pltpu.sync_copy(ones_hbm, rows_a)
    plsc.subcore_barrier()

    for p in range(NST):
        base = wid * NCH + p * STG
        pltpu.sync_copy(dst_hbm.at[pl.ds(base, STG)], dst_v)

        def group(g, carry):
            for t in range(8):
                pltpu.async_copy(
                    rows_a, acc_sh.at[dst_v.at[g * 8 + t]], sa, add=True)
            for t in range(8):
                pltpu.make_async_copy(rows_a, acc_sh.at[dst_v.at[0]], sa).wait()
            return carry

        lax.fori_loop(0, STG // 8, group, 0)

    plsc.subcore_barrier()
    pltpu.sync_copy(acc_sh.at[pl.ds(r0, RPT)], degp.at[c, pl.ds(r0, RPT)])
    plsc.subcore_barrier()
    pltpu.sync_copy(zeros_hbm, acc_sh.at[pl.ds(r0, RPT)])
    plsc.subcore_barrier()
    _seg_core(x_hbm, src_hbm, dst_hbm, parts,
              acc_sh, src_v, dst_v, rows_a, rows_b, ga, gb, sa, sb,
              c, s, wid, r0)


_sc_segdeg = functools.partial(
    pl.kernel,
    out_type=[
        jax.ShapeDtypeStruct((NC, NPAD, DW), jnp.float32),
        jax.ShapeDtypeStruct((NC, NPAD, D), jnp.float32),
    ],
    mesh=_mesh,
    scratch_types=list(_SEG_SCRATCH),
)(_segdeg_body)


def _tc_body(relu, parts_ref, degp_ref, h_ref, wlT_ref, wrT_ref, b_ref, out_ref):
    ssum = parts_ref[0] + parts_ref[1]
    deg = degp_ref[0, :, 0:1] + degp_ref[1, :, 0:1]
    agg = ssum / jnp.maximum(deg, 1.0)
    out = jnp.dot(agg, wlT_ref[...], preferred_element_type=jnp.float32)
    out = out + jnp.dot(h_ref[...], wrT_ref[...], preferred_element_type=jnp.float32)
    out = out + b_ref[...]
    if relu:
        out = jnp.maximum(out, 0.0)
    out_ref[...] = out


_BN = 2000


def _tc_layer(parts, degp, h, wlT, wrT, b, relu):
    grid = (N // _BN,)
    return pl.pallas_call(
        functools.partial(_tc_body, relu),
        grid=grid,
        in_specs=[
            pl.BlockSpec((NC, _BN, D), lambda i: (0, i, 0)),
            pl.BlockSpec((NC, _BN, DW), lambda i: (0, i, 0)),
            pl.BlockSpec((_BN, D), lambda i: (i, 0)),
            pl.BlockSpec((D, D), lambda i: (0, 0)),
            pl.BlockSpec((D, D), lambda i: (0, 0)),
            pl.BlockSpec((1, D), lambda i: (0, 0)),
        ],
        out_specs=pl.BlockSpec((_BN, D), lambda i: (i, 0)),
        out_shape=jax.ShapeDtypeStruct((N, D), jnp.float32),
    )(parts, degp, h, wlT, wrT, b)


def kernel(x, edge_index, Wl1, bl1, Wr1, Wl2, bl2, Wr2, Wl3, bl3, Wr3,
           g1, be1, g2, be2):
    src = edge_index[0].astype(jnp.int32).reshape(NW * NCH, K)
    dst = edge_index[1].astype(jnp.int32).reshape(NW * NCH, K)
    zeros = jnp.zeros((RPT, D), jnp.float32)
    ones = jnp.ones((K, D), jnp.float32)

    gs1 = g1 / jnp.sqrt(1.0 + EPS)
    gs2 = g2 / jnp.sqrt(1.0 + EPS)
    wlT1 = Wl1.T * gs1[None, :]
    wrT1 = Wr1.T * gs1[None, :]
    b1 = (bl1 * gs1 + be1)[None, :]
    wlT2 = Wl2.T * gs2[None, :]
    wrT2 = Wr2.T * gs2[None, :]
    b2 = (bl2 * gs2 + be2)[None, :]
    wlT3 = Wl3.T
    wrT3 = Wr3.T
    b3 = bl3[None, :]

    degp, parts = _sc_segdeg(x, src, dst, zeros, ones)
    h = _tc_layer(parts, degp, x, wlT1, wrT1, b1, True)
    parts = _sc_seg(h, src, dst, zeros)
    h = _tc_layer(parts, degp, h, wlT2, wrT2, b2, True)
    parts = _sc_seg(h, src, dst, zeros)
    h = _tc_layer(parts, degp, h, wlT3, wrT3, b3, False)
    return h
